# baseline (speedup 1.0000x reference)
import functools

import jax
import jax.numpy as jnp
from jax import lax
from jax.experimental import pallas as pl
from jax.experimental.pallas import tpu as pltpu
from jax.experimental.pallas import tpu_sc as plsc

E = 320000
N_PTS = 10000
N_VIEWS = 500
NVP = 512
D = 128

NC = 2
NS = 16
NW = NC * NS
BPW = E // NW
C = 200

BE = 8000
NB = E // BE


def _sc_gather_pt(pt_tbl, pt_idx):
    mesh = plsc.VectorSubcoreMesh(core_axis_name="c", subcore_axis_name="s")

    @functools.partial(
        pl.kernel,
        mesh=mesh,
        out_type=jax.ShapeDtypeStruct((E, D), jnp.float32),
        scratch_types=[
            pltpu.VMEM((C,), jnp.int32),
            pltpu.VMEM((C,), jnp.int32),
            pltpu.VMEM((C, D), jnp.float32),
            pltpu.VMEM((C, D), jnp.float32),
            pltpu.SemaphoreType.DMA,
            pltpu.SemaphoreType.DMA,
            pltpu.SemaphoreType.DMA,
            pltpu.SemaphoreType.DMA,
        ],
    )
    def k(pt_hbm, pi_hbm, o_hbm, piA, piB, rpA, rpB, gsA, gsB, ssA, ssB):
        wid = lax.axis_index("s") * NC + lax.axis_index("c")
        base = wid * BPW

        pltpu.sync_copy(pi_hbm.at[pl.ds(base, C)], piA)
        pltpu.async_copy(pt_hbm.at[piA], rpA, gsA)

        @pl.loop(0, BPW, step=2 * C)
        def _(off):
            sA = base + off
            sB = sA + C
            pltpu.make_async_copy(pt_hbm.at[piA], rpA, gsA).wait()

            @pl.when(off > 0)
            def _():
                pltpu.make_async_copy(rpB, o_hbm.at[pl.ds(sB - 2 * C, C)],
                                      ssB).wait()

            pltpu.sync_copy(pi_hbm.at[pl.ds(sB, C)], piB)
            pltpu.async_copy(pt_hbm.at[piB], rpB, gsB)
            pltpu.async_copy(rpA, o_hbm.at[pl.ds(sA, C)], ssA)

            pltpu.make_async_copy(pt_hbm.at[piB], rpB, gsB).wait()
            pltpu.make_async_copy(rpA, o_hbm.at[pl.ds(sA, C)], ssA).wait()

            @pl.when(off + 2 * C < BPW)
            def _():
                pltpu.sync_copy(pi_hbm.at[pl.ds(sB + C, C)], piA)
                pltpu.async_copy(pt_hbm.at[piA], rpA, gsA)

            pltpu.async_copy(rpB, o_hbm.at[pl.ds(sB, C)], ssB)

        pltpu.make_async_copy(rpB, o_hbm.at[pl.ds(base + BPW - C, C)],
                              ssB).wait()

    return k(pt_tbl, pt_idx)


def _tc_body(v_ref, p_ref, ci_ref, iot_ref, w_ref, vw_ref, bg_ref, o_ref):
    vb = v_ref[...].astype(jnp.bfloat16)
    wb = w_ref[...].astype(jnp.bfloat16)
    acc = lax.dot_general(
        vb, wb, (((1,), (1,)), ((), ())),
        preferred_element_type=jnp.float32,
    )
    cam = ci_ref[0, 0, :].astype(jnp.int16)
    oh = jnp.where(cam[:, None] == iot_ref[...],
                   jnp.bfloat16(1), jnp.bfloat16(0))
    view = lax.dot_general(
        oh, vw_ref[...], (((1,), (0,)), ((), ())),
        preferred_element_type=jnp.float32,
    )
    o_ref[...] = (acc + view + p_ref[...] + bg_ref[...]) * 0.25


def kernel(values, scenepoint_features, view_features, global_features,
           cam_idx, pt_idx, W, b):
    pt_rows = _sc_gather_pt(scenepoint_features, pt_idx.astype(jnp.int32))

    ci3 = cam_idx.astype(jnp.int32).reshape(NB, 1, BE)
    iot = lax.iota(jnp.int16, NVP)[None, :]
    vw_pad = jnp.zeros((NVP, D), jnp.bfloat16).at[:N_VIEWS].set(
        view_features.astype(jnp.bfloat16))
    bg = (b + global_features)[None, :]

    out = pl.pallas_call(
        _tc_body,
        grid=(NB,),
        in_specs=[
            pl.BlockSpec((BE, D), lambda i: (i, 0)),
            pl.BlockSpec((BE, D), lambda i: (i, 0)),
            pl.BlockSpec((1, 1, BE), lambda i: (i, 0, 0)),
            pl.BlockSpec((1, NVP), lambda i: (0, 0)),
            pl.BlockSpec((D, D), lambda i: (0, 0)),
            pl.BlockSpec((NVP, D), lambda i: (0, 0)),
            pl.BlockSpec((1, D), lambda i: (0, 0)),
        ],
        out_specs=pl.BlockSpec((BE, D), lambda i: (i, 0)),
        out_shape=jax.ShapeDtypeStruct((E, D), jnp.float32),
    )(values, pt_rows, ci3, iot, W, vw_pad, bg)
    return out


# speedup vs baseline: 1.0092x; 1.0092x over previous
import functools

import jax
import jax.numpy as jnp
from jax import lax
from jax.experimental import pallas as pl
from jax.experimental.pallas import tpu as pltpu
from jax.experimental.pallas import tpu_sc as plsc

E = 320000
N_PTS = 10000
N_VIEWS = 500
NVP = 512
D = 128

NC = 2
NS = 16
NW = NC * NS
BPW = E // NW
C = 400

BE = 8000
NB = E // BE


def _sc_gather_pt(pt_tbl, pt_idx):
    mesh = plsc.VectorSubcoreMesh(core_axis_name="c", subcore_axis_name="s")

    @functools.partial(
        pl.kernel,
        mesh=mesh,
        out_type=jax.ShapeDtypeStruct((E, D), jnp.float32),
        scratch_types=[
            pltpu.VMEM((C,), jnp.int32),
            pltpu.VMEM((C, D), jnp.float32),
            pltpu.SemaphoreType.DMA,
        ],
    )
    def k(pt_hbm, pi_hbm, o_hbm, pi_v, rp_v, sem):
        wid = lax.axis_index("s") * NC + lax.axis_index("c")
        base = wid * BPW

        @pl.loop(0, BPW, step=C)
        def _(off):
            s = base + off
            pltpu.sync_copy(pi_hbm.at[pl.ds(s, C)], pi_v)
            pltpu.async_copy(pt_hbm.at[pi_v], rp_v, sem).wait()
            pltpu.sync_copy(rp_v, o_hbm.at[pl.ds(s, C)])

    return k(pt_tbl, pt_idx)


def _tc_body(v_ref, p_ref, ci_ref, iot_ref, w_ref, vw_ref, bg_ref, o_ref):
    vb = v_ref[...].astype(jnp.bfloat16)
    wb = w_ref[...].astype(jnp.bfloat16)
    acc = lax.dot_general(
        vb, wb, (((1,), (1,)), ((), ())),
        preferred_element_type=jnp.float32,
    )
    cam = ci_ref[0, 0, :].astype(jnp.int16)
    oh = jnp.where(cam[:, None] == iot_ref[...],
                   jnp.bfloat16(1), jnp.bfloat16(0))
    view = lax.dot_general(
        oh, vw_ref[...], (((1,), (0,)), ((), ())),
        preferred_element_type=jnp.float32,
    )
    o_ref[...] = (acc + view + p_ref[...] + bg_ref[...]) * 0.25


def kernel(values, scenepoint_features, view_features, global_features,
           cam_idx, pt_idx, W, b):
    pt_rows = _sc_gather_pt(scenepoint_features, pt_idx.astype(jnp.int32))

    ci3 = cam_idx.astype(jnp.int32).reshape(NB, 1, BE)
    iot = lax.iota(jnp.int16, NVP)[None, :]
    vw_pad = jnp.zeros((NVP, D), jnp.bfloat16).at[:N_VIEWS].set(
        view_features.astype(jnp.bfloat16))
    bg = (b + global_features)[None, :]

    out = pl.pallas_call(
        _tc_body,
        grid=(NB,),
        in_specs=[
            pl.BlockSpec((BE, D), lambda i: (i, 0)),
            pl.BlockSpec((BE, D), lambda i: (i, 0)),
            pl.BlockSpec((1, 1, BE), lambda i: (i, 0, 0)),
            pl.BlockSpec((1, NVP), lambda i: (0, 0)),
            pl.BlockSpec((D, D), lambda i: (0, 0)),
            pl.BlockSpec((NVP, D), lambda i: (0, 0)),
            pl.BlockSpec((1, D), lambda i: (0, 0)),
        ],
        out_specs=pl.BlockSpec((BE, D), lambda i: (i, 0)),
        out_shape=jax.ShapeDtypeStruct((E, D), jnp.float32),
    )(values, pt_rows, ci3, iot, W, vw_pad, bg)
    return out
